# baseline (speedup 1.0000x reference)
import functools

import jax
import jax.numpy as jnp
from jax import lax
from jax.experimental import pallas as pl
from jax.experimental.pallas import tpu as pltpu
from jax.experimental.pallas import tpu_sc as plsc

_B = 16384
_F = 26
_D = 16
_IN = _F * _D
_FP = 32
_V = 1000000

_NC = 2
_NS = 16
_NW = _NC * _NS
_TOT = _B * _FP
_RPW = _TOT // _NW
_CHUNK = 128
_NCH = _RPW // _CHUNK
_ORPC = _CHUNK * _D // 128
_ORPW = _NCH * _ORPC


def _sc_gather_body(table_hbm, idx_hbm, out_hbm, idx_v, bufa, bufb, sema, semb):
    table2d = table_hbm
    wid = lax.axis_index("s") * _NC + lax.axis_index("c")
    pltpu.sync_copy(idx_hbm.at[pl.ds(wid * _NCH, _NCH)], idx_v)
    out_rbase = wid * _RPW
    pltpu.async_copy(table2d.at[idx_v.at[0]], bufa, sema)

    def step(jj, carry):
        j0 = 2 * jj
        j1 = j0 + 1
        pltpu.async_copy(table2d.at[idx_v.at[j1]], bufb, semb)
        pltpu.make_async_copy(table2d.at[idx_v.at[j0]], bufa, sema).wait()
        pltpu.sync_copy(bufa, out_hbm.at[pl.ds(out_rbase + j0 * _CHUNK, _CHUNK)])

        @pl.when(jj + 1 < _NCH // 2)
        def _():
            pltpu.async_copy(table2d.at[idx_v.at[j0 + 2]], bufa, sema)

        pltpu.make_async_copy(table2d.at[idx_v.at[j1]], bufb, semb).wait()
        pltpu.sync_copy(bufb, out_hbm.at[pl.ds(out_rbase + j1 * _CHUNK, _CHUNK)])
        return carry

    lax.fori_loop(0, _NCH // 2, step, 0)


@jax.jit
def _sc_gather(table, idx2d):
    mesh = plsc.VectorSubcoreMesh(core_axis_name="c", subcore_axis_name="s")
    return pl.kernel(
        _sc_gather_body,
        out_type=jax.ShapeDtypeStruct((_TOT, _D), jnp.float32),
        mesh=mesh,
        scratch_types=[
            pltpu.VMEM((_NCH, _CHUNK), jnp.int32),
            pltpu.VMEM((_CHUNK, _D), jnp.float32),
            pltpu.VMEM((_CHUNK, _D), jnp.float32),
            pltpu.SemaphoreType.DMA,
            pltpu.SemaphoreType.DMA,
        ],
        compiler_params=pltpu.CompilerParams(use_tc_tiling_on_sc=False),
    )(table, idx2d)


_BS = 512


def _tc_body(x0, x1, x2, x3, sid_ref, hw1, hb1, hw2, hb2, dmw, dmb,
             sw1, sb1, sw2, sb2, exw, exb, gw, gb, twd, twb, pd, pdb, sgw,
             o0, o1, o2, om):
    x = jnp.concatenate([x0[...], x1[...], x2[...], x3[...]], axis=1)
    x = x[:, :_IN]
    sid = sid_ref[...]

    h = jnp.maximum(jnp.dot(x, hw1[...], preferred_element_type=jnp.float32)
                    + hb1[...], 0.0)
    h = jnp.maximum(jnp.dot(h, hw2[...], preferred_element_type=jnp.float32)
                    + hb2[...], 0.0)
    m = jnp.dot(h, dmw[...], preferred_element_type=jnp.float32) + dmb[...]
    s = jax.nn.sigmoid(m)
    oh = (lax.broadcasted_iota(jnp.int32, (1, 3), 1) == sid).astype(jnp.float32)
    ssel = jnp.sum(s * oh, axis=1, keepdims=True)
    om[...] = jnp.sign(jnp.maximum(ssel - 0.5, 0.0))

    sh = jnp.maximum(jnp.dot(x, sw1[...], preferred_element_type=jnp.float32)
                     + sb1[...], 0.0)
    sh = jnp.maximum(jnp.dot(sh, sw2[...], preferred_element_type=jnp.float32)
                     + sb2[...], 0.0)
    eo = jnp.maximum(jnp.dot(sh, exw[...], preferred_element_type=jnp.float32)
                     + exb[...], 0.0)
    gl = jnp.dot(sh, gw[...], preferred_element_type=jnp.float32) + gb[...]
    wlog = jnp.dot(sh, sgw[...], preferred_element_type=jnp.float32)

    mixes = []
    for i in range(3):
        gi = gl[:, 8 * i:8 * (i + 1)]
        gi = gi - jnp.max(gi, axis=1, keepdims=True)
        egi = jnp.exp(gi)
        g = egi / jnp.sum(egi, axis=1, keepdims=True)
        mix = g[:, 0:1] * eo[:, 0:64]
        for e in range(1, 8):
            mix = mix + g[:, e:e + 1] * eo[:, 64 * e:64 * (e + 1)]
        mixes.append(mix)
    mixcat = jnp.concatenate(mixes, axis=1)
    t = jnp.maximum(jnp.dot(mixcat, twd[...],
                            preferred_element_type=jnp.float32) + twb[...],
                    0.0)
    sc = jax.nn.sigmoid(jnp.dot(t, pd[...], preferred_element_type=jnp.float32)
                        + pdb[...])

    for i, oref in enumerate((o0, o1, o2)):
        wi = wlog[:, 3 * i:3 * (i + 1)]
        wi = wi - jnp.max(wi, axis=1, keepdims=True)
        ew = jnp.exp(wi)
        w = ew / jnp.sum(ew, axis=1, keepdims=True)
        oref[...] = jnp.sum(w * sc, axis=1)


def _full(shape):
    nd = len(shape)
    return pl.BlockSpec(shape, lambda i, _nd=nd: (0,) * _nd)


def _xspec(c):
    nblk = _B // _BS
    return pl.BlockSpec((_BS, 128), lambda i, _c=c, _n=nblk: (_c * _n + i, 0))


@jax.jit
def _tc_dense(feat128, sid, hw1, hb1, hw2, hb2, dmw, dmb,
              sw1, sb1, sw2, sb2, exw, exb, gw, gb, twd, twb, pd, pdb, sgw):
    grid = _B // _BS
    return pl.pallas_call(
        _tc_body,
        grid=(grid,),
        in_specs=[
            _xspec(0), _xspec(1), _xspec(2), _xspec(3),
            pl.BlockSpec((_BS, 1), lambda i: (i, 0)),
            _full((_IN, 128)), _full((128,)),
            _full((128, 64)), _full((64,)),
            _full((64, 3)), _full((1, 3)),
            _full((_IN, 256)), _full((256,)),
            _full((256, 128)), _full((128,)),
            _full((128, 512)), _full((512,)),
            _full((128, 24)), _full((24,)),
            _full((192, 192)), _full((192,)),
            _full((192, 3)), _full((1, 3)),
            _full((128, 9)),
        ],
        out_specs=[
            pl.BlockSpec((_BS,), lambda i: (i,)),
            pl.BlockSpec((_BS,), lambda i: (i,)),
            pl.BlockSpec((_BS,), lambda i: (i,)),
            pl.BlockSpec((_BS, 1), lambda i: (i, 0)),
        ],
        out_shape=[
            jax.ShapeDtypeStruct((_B,), jnp.float32),
            jax.ShapeDtypeStruct((_B,), jnp.float32),
            jax.ShapeDtypeStruct((_B,), jnp.float32),
            jax.ShapeDtypeStruct((_B, 1), jnp.float32),
        ],
    )(feat128, feat128, feat128, feat128, sid,
      hw1, hb1, hw2, hb2, dmw, dmb,
      sw1, sb1, sw2, sb2, exw, exb, gw, gb, twd, twb, pd, pdb, sgw)


def kernel(inputs, label, sid, emb_table, dom_emb_table, share_W1, share_b1,
           share_W2, share_b2, expert_W, expert_b, gate_W, gate_b, tower_W,
           tower_b, pred_W, pred_b, sg_W, hyper_W1, hyper_b1, hyper_W2,
           hyper_b2, dm_W, dm_b):
    idxp = jnp.concatenate(
        [inputs.astype(jnp.int32), jnp.zeros((_B, _FP - _F), jnp.int32)], axis=1)
    idx4 = idxp.reshape(_B, 4, 8).transpose(1, 0, 2).reshape(_TOT // _CHUNK,
                                                             _CHUNK)
    feat128 = _sc_gather(emb_table, idx4).reshape(_TOT * _D // 128, 128)

    dmw = dm_W[:, :, 0].T
    dmb = dm_b.reshape(1, 3)
    exw = expert_W.transpose(1, 0, 2).reshape(128, 512)
    exb = expert_b.reshape(512)
    gw = gate_W.transpose(1, 0, 2).reshape(128, 24)
    gb = gate_b.reshape(24)
    twd = jnp.zeros((192, 192), jnp.float32)
    pd = jnp.zeros((192, 3), jnp.float32)
    for i in range(3):
        twd = twd.at[64 * i:64 * (i + 1), 64 * i:64 * (i + 1)].set(tower_W[i])
        pd = pd.at[64 * i:64 * (i + 1), i].set(pred_W[i, :, 0])
    twb = tower_b.reshape(192)
    pdb = pred_b.reshape(1, 3)
    sgw = sg_W.transpose(1, 0, 2).reshape(128, 9)

    o0, o1, o2, smask = _tc_dense(
        feat128, sid, hyper_W1, hyper_b1, hyper_W2, hyper_b2, dmw, dmb,
        share_W1, share_b1, share_W2, share_b2, exw, exb, gw, gb,
        twd, twb, pd, pdb, sgw)
    return (o0, o1, o2, sid, label, smask)
